# baseline (speedup 1.0000x reference)
import jax
import jax.numpy as jnp
from jax.experimental import pallas as pl


def _moe_body(x_ref, gw_ref, w_ref, em_ref, ow_ref, ob_ref,
              gates_ref, y_ref, emo_ref):
    h = x_ref[:, 0, :]
    ow = ow_ref[...]
    b = ob_ref[0, 0]

    gates_ref[...] = jax.lax.dot_general(
        h, gw_ref[...], (((1,), (1,)), ((), ())),
        preferred_element_type=jnp.float32)

    vmin = jax.lax.dot_general(
        em_ref[...], ow, (((1,), (1,)), ((), ())),
        preferred_element_type=jnp.float32)
    emo_ref[...] = jax.lax.dot_general(
        h, vmin, (((1,), (0,)), ((), ()))) + b

    v = jnp.sum(w_ref[...] * ow[None, :, :], axis=2)

    y_ref[...] = jax.lax.dot_general(
        h, v, (((1,), (1,)), ((), ())),
        preferred_element_type=jnp.float32) + b


def kernel(x, gate_weights, expert_weights, expert_min, out_w, out_b):
    B, _, H = x.shape
    E = expert_weights.shape[0]
    ob2 = out_b.reshape(1, 1)

    gates, y2, emo = pl.pallas_call(
        _moe_body,
        out_shape=[
            jax.ShapeDtypeStruct((B, E), jnp.float32),
            jax.ShapeDtypeStruct((B, E), jnp.float32),
            jax.ShapeDtypeStruct((B, 1), jnp.float32),
        ],
    )(x, gate_weights, expert_weights, expert_min, out_w, ob2)

    return (gates, y2.reshape(B, E, 1), emo)


# speedup vs baseline: 1.0167x; 1.0167x over previous
import jax
import jax.numpy as jnp
from jax.experimental import pallas as pl
from jax.experimental.pallas import tpu as pltpu

NCHUNK = 8


def _moe_body(x_hbm, gw_ref, w_hbm, em_ref, ow_ref, ob_ref,
              gates_ref, y_ref, emo_ref, h_vmem, w_vmem, sems):
    E = w_vmem.shape[0]
    ce = E // NCHUNK
    hcp = pltpu.make_async_copy(x_hbm.at[:, 0, :], h_vmem, sems.at[NCHUNK])
    hcp.start()
    hcp.wait()

    wcopies = [
        pltpu.make_async_copy(
            w_hbm.at[pl.ds(k * ce, ce)], w_vmem.at[pl.ds(k * ce, ce)],
            sems.at[k])
        for k in range(NCHUNK)
    ]
    for c in wcopies:
        c.start()

    ow = ow_ref[...]
    b = ob_ref[0, 0]
    h = h_vmem[...]

    gates_ref[...] = jax.lax.dot_general(
        h, gw_ref[...], (((1,), (1,)), ((), ())),
        preferred_element_type=jnp.float32)

    vmin = jax.lax.dot_general(
        em_ref[...], ow, (((1,), (1,)), ((), ())),
        preferred_element_type=jnp.float32)
    emo_ref[...] = jax.lax.dot_general(
        h, vmin, (((1,), (0,)), ((), ()))) + b

    vparts = []
    for k, c in enumerate(wcopies):
        c.wait()
        vparts.append(
            jnp.sum(w_vmem[pl.ds(k * ce, ce)] * ow[None, :, :], axis=2))
    v = jnp.concatenate(vparts, axis=0)

    y_ref[...] = jax.lax.dot_general(
        h, v, (((1,), (1,)), ((), ())),
        preferred_element_type=jnp.float32) + b


def kernel(x, gate_weights, expert_weights, expert_min, out_w, out_b):
    B, _, H = x.shape
    E, _, S = expert_weights.shape
    ob2 = out_b.reshape(1, 1)

    gates, y2, emo = pl.pallas_call(
        _moe_body,
        in_specs=[
            pl.BlockSpec(memory_space=pltpu.MemorySpace.HBM),
            pl.BlockSpec(memory_space=pltpu.VMEM),
            pl.BlockSpec(memory_space=pltpu.MemorySpace.HBM),
            pl.BlockSpec(memory_space=pltpu.VMEM),
            pl.BlockSpec(memory_space=pltpu.VMEM),
            pl.BlockSpec(memory_space=pltpu.VMEM),
        ],
        out_shape=[
            jax.ShapeDtypeStruct((B, E), jnp.float32),
            jax.ShapeDtypeStruct((B, E), jnp.float32),
            jax.ShapeDtypeStruct((B, 1), jnp.float32),
        ],
        scratch_shapes=[
            pltpu.VMEM((B, H), jnp.float32),
            pltpu.VMEM((E, H, S), jnp.float32),
            pltpu.SemaphoreType.DMA((NCHUNK + 1,)),
        ],
    )(x, gate_weights, expert_weights, expert_min, out_w, ob2)

    return (gates, y2.reshape(B, E, 1), emo)


# speedup vs baseline: 1.0230x; 1.0063x over previous
import jax
import jax.numpy as jnp
from jax.experimental import pallas as pl
from jax.experimental.pallas import tpu as pltpu

NCHUNK = 8


def _moe_body(x_hbm, gw_ref, w_hbm, em_ref, ow_ref, ob_ref,
              gates_ref, y_ref, emo_ref, h_vmem, w_vmem, x_vmem, sems):
    E = w_vmem.shape[0]
    ce = E // NCHUNK

    xcp = pltpu.make_async_copy(x_hbm, x_vmem, sems.at[NCHUNK])
    xcp.start()
    wcopies = [
        pltpu.make_async_copy(
            w_hbm.at[pl.ds(k * ce, ce)], w_vmem.at[pl.ds(k * ce, ce)],
            sems.at[k])
        for k in range(NCHUNK)
    ]
    for c in wcopies:
        c.start()

    ow = ow_ref[...]
    b = ob_ref[0, 0]

    vmin = jax.lax.dot_general(
        em_ref[...], ow, (((1,), (1,)), ((), ())),
        preferred_element_type=jnp.float32)

    xcp.wait()
    h_vmem[...] = x_vmem[:, 0, :]
    h = h_vmem[...]

    gates_ref[...] = jax.lax.dot_general(
        h, gw_ref[...], (((1,), (1,)), ((), ())),
        preferred_element_type=jnp.float32)

    emo_ref[...] = jax.lax.dot_general(
        h, vmin, (((1,), (0,)), ((), ()))) + b

    vparts = []
    for k, c in enumerate(wcopies):
        c.wait()
        vparts.append(
            jnp.sum(w_vmem[pl.ds(k * ce, ce)] * ow[None, :, :], axis=2))
    v = jnp.concatenate(vparts, axis=0)

    y_ref[...] = jax.lax.dot_general(
        h, v, (((1,), (1,)), ((), ())),
        preferred_element_type=jnp.float32) + b


def kernel(x, gate_weights, expert_weights, expert_min, out_w, out_b):
    B, _, H = x.shape
    E, _, S = expert_weights.shape
    ob2 = out_b.reshape(1, 1)

    gates, y2, emo = pl.pallas_call(
        _moe_body,
        in_specs=[
            pl.BlockSpec(memory_space=pltpu.MemorySpace.HBM),
            pl.BlockSpec(memory_space=pltpu.VMEM),
            pl.BlockSpec(memory_space=pltpu.MemorySpace.HBM),
            pl.BlockSpec(memory_space=pltpu.VMEM),
            pl.BlockSpec(memory_space=pltpu.VMEM),
            pl.BlockSpec(memory_space=pltpu.VMEM),
        ],
        out_shape=[
            jax.ShapeDtypeStruct((B, E), jnp.float32),
            jax.ShapeDtypeStruct((B, E), jnp.float32),
            jax.ShapeDtypeStruct((B, 1), jnp.float32),
        ],
        scratch_shapes=[
            pltpu.VMEM((B, H), jnp.float32),
            pltpu.VMEM((E, H, S), jnp.float32),
            pltpu.VMEM((B, 1, H), jnp.float32),
            pltpu.SemaphoreType.DMA((NCHUNK + 1,)),
        ],
    )(x, gate_weights, expert_weights, expert_min, out_w, ob2)

    return (gates, y2.reshape(B, E, 1), emo)
